# baseline (speedup 1.0000x reference)
import functools

import jax
import jax.numpy as jnp
from jax import lax
from jax.experimental import pallas as pl
from jax.experimental.pallas import tpu as pltpu
from jax.experimental.pallas import tpu_sc as plsc

D_MODEL = 768
D_B = 64
D_PAD = 128
L_PAD = 24
POOL_D = D_B + 2 * D_MODEL
OUT_D = 3 * POOL_D

_NC = 2
_NS = 16
_NW = _NC * _NS
_PH = 2


@functools.lru_cache(maxsize=None)
def _make_kernel(B: int, L: int):
    rows_per_w = B // _NW
    rows_per_ph = rows_per_w // _PH
    mesh = plsc.VectorSubcoreMesh(core_axis_name="c", subcore_axis_name="s")

    out_types = tuple(
        jax.ShapeDtypeStruct((B, L_PAD, d), jnp.float32)
        for d in (D_PAD, D_MODEL, D_MODEL) * 3
    )

    @functools.partial(
        pl.kernel,
        mesh=mesh,
        out_type=out_types,
        compiler_params=pltpu.CompilerParams(use_tc_tiling_on_sc=True),
        scratch_types=[
            pltpu.VMEM((rows_per_ph, L_PAD), jnp.int32),
            pltpu.VMEM((rows_per_ph, L_PAD), jnp.int32),
            pltpu.VMEM((rows_per_ph, L_PAD), jnp.int32),
            pltpu.VMEM((L_PAD, D_PAD), jnp.float32),
            pltpu.VMEM((L_PAD, D_MODEL), jnp.float32),
            pltpu.VMEM((L_PAD, D_MODEL), jnp.float32),
            pltpu.VMEM((L_PAD, D_PAD), jnp.float32),
            pltpu.VMEM((L_PAD, D_MODEL), jnp.float32),
            pltpu.VMEM((L_PAD, D_MODEL), jnp.float32),
            pltpu.VMEM((L_PAD, D_PAD), jnp.float32),
            pltpu.VMEM((L_PAD, D_MODEL), jnp.float32),
            pltpu.VMEM((L_PAD, D_MODEL), jnp.float32),
            pltpu.SemaphoreType.DMA,
        ],
    )
    def k(qk_idx, v_idx, know_idx,
          qk_emb, v_emb, know_emb,
          qk_read, v_read, know_read,
          qk_write, v_write, know_write,
          o_qe, o_qr, o_qw, o_ve, o_vr, o_vw, o_ke, o_kr, o_kw,
          qk_iv, v_iv, know_iv,
          b_qe, b_qr, b_qw, b_ve, b_vr, b_vw, b_ke, b_kr, b_kw,
          sem):
        wid = lax.axis_index("s") * _NC + lax.axis_index("c")
        base_b = wid * rows_per_w

        jobs = [
            (qk_iv, qk_emb, b_qe, o_qe),
            (qk_iv, qk_read, b_qr, o_qr),
            (qk_iv, qk_write, b_qw, o_qw),
            (v_iv, v_emb, b_ve, o_ve),
            (v_iv, v_read, b_vr, o_vr),
            (v_iv, v_write, b_vw, o_vw),
            (know_iv, know_emb, b_ke, o_ke),
            (know_iv, know_read, b_kr, o_kr),
            (know_iv, know_write, b_kw, o_kw),
        ]

        for p in range(_PH):
            slab = wid * _PH + p
            pltpu.sync_copy(qk_idx.at[slab], qk_iv)
            pltpu.sync_copy(v_idx.at[slab], v_iv)
            pltpu.sync_copy(know_idx.at[slab], know_iv)

            def body(j, carry):
                b = base_b + p * rows_per_ph + j
                copies = [pltpu.async_copy(tab.at[iv.at[j]], buf, sem)
                          for (iv, tab, buf, _o) in jobs]
                for c in copies:
                    c.wait()
                for (_iv, _tab, buf, o) in jobs:
                    pltpu.sync_copy(buf, o.at[b])
                return carry

            lax.fori_loop(0, rows_per_ph, body, 0)

    return k


def kernel(qk_idx, v_idx, know_idx, qk_emb, v_emb, know_emb,
           qk_read, v_read, know_read, qk_write, v_write, know_write):
    B, L = qk_idx.shape
    shape = (_NW * _PH, B // _NW // _PH, L)
    pad = ((0, 0), (0, D_PAD - D_B))
    ipad = ((0, 0), (0, 0), (0, L_PAD - L))
    outs = _make_kernel(B, L)(
        jnp.pad(qk_idx.reshape(shape), ipad),
        jnp.pad(v_idx.reshape(shape), ipad),
        jnp.pad(know_idx.reshape(shape), ipad),
        jnp.pad(qk_emb, pad), jnp.pad(v_emb, pad), jnp.pad(know_emb, pad),
        qk_read, v_read, know_read,
        qk_write, v_write, know_write)
    pieces = [o[:, :L, :D_B] if i % 3 == 0 else o[:, :L, :]
              for i, o in enumerate(outs)]
    return jnp.concatenate(pieces, axis=-1)


# speedup vs baseline: 1.0119x; 1.0119x over previous
import functools

import jax
import jax.numpy as jnp
from jax import lax
from jax.experimental import pallas as pl
from jax.experimental.pallas import tpu as pltpu
from jax.experimental.pallas import tpu_sc as plsc

D_MODEL = 768
D_B = 64
D_PAD = 128
L_PAD = 24
POOL_D = D_B + 2 * D_MODEL
OUT_D = 3 * POOL_D

_NC = 2
_NS = 16
_NW = _NC * _NS

_BB = 128
_LANES = 16


@functools.lru_cache(maxsize=None)
def _make_kernel(B: int, L: int):
    n_bblk = B // _BB
    l_per_w = L * n_bblk // _NW
    mesh = plsc.VectorSubcoreMesh(core_axis_name="c", subcore_axis_name="s")

    @functools.partial(
        pl.kernel,
        mesh=mesh,
        out_type=jax.ShapeDtypeStruct((L, OUT_D, B), jnp.float32),
        compiler_params=pltpu.CompilerParams(
            use_tc_tiling_on_sc=True, needs_layout_passes=False),
        scratch_types=[
            pltpu.VMEM((L_PAD, _BB), jnp.int32),
            pltpu.VMEM((L_PAD, _BB), jnp.int32),
            pltpu.VMEM((L_PAD, _BB), jnp.int32),
            pltpu.VMEM((_BB, D_MODEL), jnp.float32),
            pltpu.VMEM((_BB, _BB), jnp.float32),
            pltpu.SemaphoreType.DMA,
        ],
    )
    def k(qk_idx, v_idx, know_idx,
          qk_emb, v_emb, know_emb,
          qk_read, v_read, know_read,
          qk_write, v_write, know_write,
          out,
          qk_iv, v_iv, know_iv,
          buf, slab,
          sem):
        wid = lax.axis_index("s") * _NC + lax.axis_index("c")
        b0 = (wid % n_bblk) * _BB
        l0 = (wid // n_bblk) * l_per_w

        pltpu.sync_copy(qk_idx.at[:, pl.ds(b0, _BB)], qk_iv)
        pltpu.sync_copy(v_idx.at[:, pl.ds(b0, _BB)], v_iv)
        pltpu.sync_copy(know_idx.at[:, pl.ds(b0, _BB)], know_iv)

        iota = lax.iota(jnp.int32, _LANES)
        dvecs = [iota + k16 * _LANES for k16 in range(_BB // _LANES)]

        jobs = [
            (qk_iv, qk_emb, 0, D_B),
            (qk_iv, qk_read, D_B, D_MODEL),
            (qk_iv, qk_write, D_B + D_MODEL, D_MODEL),
            (v_iv, v_emb, POOL_D, D_B),
            (v_iv, v_read, POOL_D + D_B, D_MODEL),
            (v_iv, v_write, POOL_D + D_B + D_MODEL, D_MODEL),
            (know_iv, know_emb, 2 * POOL_D, D_B),
            (know_iv, know_read, 2 * POOL_D + D_B, D_MODEL),
            (know_iv, know_write, 2 * POOL_D + D_B + D_MODEL, D_MODEL),
        ]

        def do_l(i, carry):
            l = l0 + i
            for (iv, tab, d_out, width) in jobs:
                dst = buf if width == D_MODEL else buf.at[:, pl.ds(0, D_PAD)]
                pltpu.async_copy(tab.at[iv.at[l]], dst, sem).wait()

                n_chunk = (width + _BB - 1) // _BB
                chunk_w = min(width, _BB)

                def do_chunk(c, carry2):
                    c0 = c * _BB

                    @plsc.parallel_loop(0, _BB, unroll=4)
                    def do_b(bp):
                        bs = jnp.full((_LANES,), bp, dtype=jnp.int32)
                        for k16 in range(chunk_w // _LANES):
                            v = buf[bp, pl.ds(c0 + k16 * _LANES, _LANES)]
                            plsc.store_scatter(slab, [dvecs[k16], bs], v)

                    pltpu.sync_copy(
                        slab.at[pl.ds(0, chunk_w)],
                        out.at[l, pl.ds(d_out + c0, chunk_w), pl.ds(b0, _BB)])
                    return carry2

                lax.fori_loop(0, n_chunk, do_chunk, 0)
            return carry

        lax.fori_loop(0, l_per_w, do_l, 0)

    return k


def kernel(qk_idx, v_idx, know_idx, qk_emb, v_emb, know_emb,
           qk_read, v_read, know_read, qk_write, v_write, know_write):
    B, L = qk_idx.shape
    ipad = ((0, 0), (0, L_PAD - L))
    epad = ((0, 0), (0, D_PAD - D_B))
    out_p = _make_kernel(B, L)(
        jnp.pad(qk_idx, ipad).T, jnp.pad(v_idx, ipad).T,
        jnp.pad(know_idx, ipad).T,
        jnp.pad(qk_emb, epad), jnp.pad(v_emb, epad), jnp.pad(know_emb, epad),
        qk_read, v_read, know_read,
        qk_write, v_write, know_write)
    return jnp.transpose(out_p, (2, 0, 1))


# speedup vs baseline: 1.4524x; 1.4353x over previous
import functools

import jax
import jax.numpy as jnp
from jax import lax
from jax.experimental import pallas as pl
from jax.experimental.pallas import tpu as pltpu
from jax.experimental.pallas import tpu_sc as plsc

D_MODEL = 768
D_B = 64
D_PAD = 128
L_PAD = 24
POOL_D = D_B + 2 * D_MODEL
OUT_D = 3 * POOL_D

_NC = 2
_NS = 16
_NW = _NC * _NS

_BB = 128
_LANES = 16


@functools.lru_cache(maxsize=None)
def _make_kernel(B: int, L: int):
    n_bblk = B // _BB
    l_per_w = L * n_bblk // _NW
    mesh = plsc.VectorSubcoreMesh(core_axis_name="c", subcore_axis_name="s")

    @functools.partial(
        pl.kernel,
        mesh=mesh,
        out_type=jax.ShapeDtypeStruct((L, OUT_D, B), jnp.float32),
        compiler_params=pltpu.CompilerParams(
            use_tc_tiling_on_sc=True, needs_layout_passes=False),
        scratch_types=[
            pltpu.VMEM((L_PAD, _BB), jnp.int32),
            pltpu.VMEM((L_PAD, _BB), jnp.int32),
            pltpu.VMEM((L_PAD, _BB), jnp.int32),
            pltpu.VMEM((_BB, D_MODEL), jnp.float32),
            pltpu.VMEM((_BB, _BB), jnp.float32),
            pltpu.SemaphoreType.DMA,
        ],
    )
    def k(qk_idx, v_idx, know_idx,
          qk_emb, v_emb, know_emb,
          qk_read, v_read, know_read,
          qk_write, v_write, know_write,
          out,
          qk_iv, v_iv, know_iv,
          buf, slab,
          sem):
        wid = lax.axis_index("s") * _NC + lax.axis_index("c")
        b0 = (wid % n_bblk) * _BB
        l0 = (wid // n_bblk) * l_per_w

        pltpu.sync_copy(qk_idx.at[:, pl.ds(b0, _BB)], qk_iv)
        pltpu.sync_copy(v_idx.at[:, pl.ds(b0, _BB)], v_iv)
        pltpu.sync_copy(know_idx.at[:, pl.ds(b0, _BB)], know_iv)

        iota = lax.iota(jnp.int32, _LANES)

        jobs = [
            (qk_iv, qk_emb, 0, D_B),
            (qk_iv, qk_read, D_B, D_MODEL),
            (qk_iv, qk_write, D_B + D_MODEL, D_MODEL),
            (v_iv, v_emb, POOL_D, D_B),
            (v_iv, v_read, POOL_D + D_B, D_MODEL),
            (v_iv, v_write, POOL_D + D_B + D_MODEL, D_MODEL),
            (know_iv, know_emb, 2 * POOL_D, D_B),
            (know_iv, know_read, 2 * POOL_D + D_B, D_MODEL),
            (know_iv, know_write, 2 * POOL_D + D_B + D_MODEL, D_MODEL),
        ]

        def do_l(i, carry):
            l = l0 + i
            for (iv, tab, d_out, width) in jobs:
                dst = buf if width == D_MODEL else buf.at[:, pl.ds(0, D_PAD)]
                pltpu.async_copy(tab.at[iv.at[l]], dst, sem).wait()

                n_chunk = (width + _BB - 1) // _BB
                chunk_w = min(width, _BB)

                def do_chunk(c, carry2):
                    c0 = c * _BB

                    @plsc.parallel_loop(0, _BB // _LANES, unroll=1)
                    def do_tb(t_b):
                        bvec = iota + t_b * _LANES

                        def do_o(o, carry3):
                            rot = (iota + o) & (_LANES - 1)
                            for t_d in range(chunk_w // _LANES):
                                srow = rot + t_d * _LANES
                                dvec = srow + c0
                                v = plsc.load_gather(buf, [bvec, dvec])
                                plsc.store_scatter(slab, [srow, bvec], v)
                            return carry3

                        lax.fori_loop(0, _LANES, do_o, 0)

                    pltpu.sync_copy(
                        slab.at[pl.ds(0, chunk_w)],
                        out.at[l, pl.ds(d_out + c0, chunk_w), pl.ds(b0, _BB)])
                    return carry2

                lax.fori_loop(0, n_chunk, do_chunk, 0)
            return carry

        lax.fori_loop(0, l_per_w, do_l, 0)

    return k


def kernel(qk_idx, v_idx, know_idx, qk_emb, v_emb, know_emb,
           qk_read, v_read, know_read, qk_write, v_write, know_write):
    B, L = qk_idx.shape
    ipad = ((0, 0), (0, L_PAD - L))
    epad = ((0, 0), (0, D_PAD - D_B))
    out_p = _make_kernel(B, L)(
        jnp.pad(qk_idx, ipad).T, jnp.pad(v_idx, ipad).T,
        jnp.pad(know_idx, ipad).T,
        jnp.pad(qk_emb, epad), jnp.pad(v_emb, epad), jnp.pad(know_emb, epad),
        qk_read, v_read, know_read,
        qk_write, v_write, know_write)
    return jnp.transpose(out_p, (2, 0, 1))


# speedup vs baseline: 2.4360x; 1.6773x over previous
import functools

import jax
import jax.numpy as jnp
from jax import lax
from jax.experimental import pallas as pl
from jax.experimental.pallas import tpu as pltpu
from jax.experimental.pallas import tpu_sc as plsc

D_MODEL = 768
D_B = 64
D_PAD = 128
L_PAD = 24
POOL_D = D_B + 2 * D_MODEL
OUT_D = 3 * POOL_D

_NC = 2
_NS = 16
_NW = _NC * _NS

_BB = 128
_LANES = 16


@functools.lru_cache(maxsize=None)
def _make_kernel(B: int, L: int):
    n_bblk = B // _BB
    l_per_w = L * n_bblk // _NW
    mesh = plsc.VectorSubcoreMesh(core_axis_name="c", subcore_axis_name="s")

    @functools.partial(
        pl.kernel,
        mesh=mesh,
        out_type=jax.ShapeDtypeStruct((L, OUT_D, B), jnp.float32),
        compiler_params=pltpu.CompilerParams(
            use_tc_tiling_on_sc=True, needs_layout_passes=False),
        scratch_types=[
            pltpu.VMEM((L_PAD, _BB), jnp.int32),
            pltpu.VMEM((L_PAD, _BB), jnp.int32),
            pltpu.VMEM((L_PAD, _BB), jnp.int32),
            pltpu.VMEM((_BB, D_MODEL), jnp.float32),
            pltpu.VMEM((_BB, _BB), jnp.float32),
            pltpu.SemaphoreType.DMA,
        ],
    )
    def k(qk_idx, v_idx, know_idx,
          qk_emb, v_emb, know_emb,
          qk_read, v_read, know_read,
          qk_write, v_write, know_write,
          out,
          qk_iv, v_iv, know_iv,
          buf, slab,
          sem):
        wid = lax.axis_index("s") * _NC + lax.axis_index("c")
        b0 = (wid % n_bblk) * _BB
        l0 = (wid // n_bblk) * l_per_w

        pltpu.sync_copy(qk_idx.at[:, pl.ds(b0, _BB)], qk_iv)
        pltpu.sync_copy(v_idx.at[:, pl.ds(b0, _BB)], v_iv)
        pltpu.sync_copy(know_idx.at[:, pl.ds(b0, _BB)], know_iv)

        iota = lax.iota(jnp.int32, _LANES)

        jobs = [
            (qk_iv, qk_emb, 0, D_B),
            (qk_iv, qk_read, D_B, D_MODEL),
            (qk_iv, qk_write, D_B + D_MODEL, D_MODEL),
            (v_iv, v_emb, POOL_D, D_B),
            (v_iv, v_read, POOL_D + D_B, D_MODEL),
            (v_iv, v_write, POOL_D + D_B + D_MODEL, D_MODEL),
            (know_iv, know_emb, 2 * POOL_D, D_B),
            (know_iv, know_read, 2 * POOL_D + D_B, D_MODEL),
            (know_iv, know_write, 2 * POOL_D + D_B + D_MODEL, D_MODEL),
        ]

        def do_l(i, carry):
            l = l0 + i
            for (iv, tab, d_out, width) in jobs:
                dst = buf if width == D_MODEL else buf.at[:, pl.ds(0, D_PAD)]
                pltpu.async_copy(tab.at[iv.at[l]], dst, sem).wait()

                n_chunk = (width + _BB - 1) // _BB
                chunk_w = min(width, _BB)

                def do_chunk(c, carry2):
                    c0 = c * _BB

                    def do_tb(t_b, carry3):
                        bvec = iota + t_b * _LANES

                        @plsc.parallel_loop(0, _LANES, unroll=4)
                        def do_o(o):
                            rot = (iota + o) & (_LANES - 1)
                            for t_d in range(chunk_w // _LANES):
                                srow = rot + t_d * _LANES
                                dvec = srow + c0
                                v = plsc.load_gather(buf, [bvec, dvec])
                                plsc.store_scatter(slab, [srow, bvec], v)

                        return carry3

                    lax.fori_loop(0, _BB // _LANES, do_tb, 0)

                    pltpu.sync_copy(
                        slab.at[pl.ds(0, chunk_w)],
                        out.at[l, pl.ds(d_out + c0, chunk_w), pl.ds(b0, _BB)])
                    return carry2

                lax.fori_loop(0, n_chunk, do_chunk, 0)
            return carry

        lax.fori_loop(0, l_per_w, do_l, 0)

    return k


def kernel(qk_idx, v_idx, know_idx, qk_emb, v_emb, know_emb,
           qk_read, v_read, know_read, qk_write, v_write, know_write):
    B, L = qk_idx.shape
    ipad = ((0, 0), (0, L_PAD - L))
    epad = ((0, 0), (0, D_PAD - D_B))
    out_p = _make_kernel(B, L)(
        jnp.pad(qk_idx, ipad).T, jnp.pad(v_idx, ipad).T,
        jnp.pad(know_idx, ipad).T,
        jnp.pad(qk_emb, epad), jnp.pad(v_emb, epad), jnp.pad(know_emb, epad),
        qk_read, v_read, know_read,
        qk_write, v_write, know_write)
    return jnp.transpose(out_p, (2, 0, 1))


# speedup vs baseline: 2.6879x; 1.1034x over previous
import functools

import jax
import jax.numpy as jnp
from jax import lax
from jax.experimental import pallas as pl
from jax.experimental.pallas import tpu as pltpu
from jax.experimental.pallas import tpu_sc as plsc

D_MODEL = 768
D_B = 64
D_PAD = 128
L_PAD = 24
POOL_D = D_B + 2 * D_MODEL
OUT_D = 3 * POOL_D

_NC = 2
_NS = 16
_NW = _NC * _NS

_BB = 128
_LANES = 16


@functools.lru_cache(maxsize=None)
def _make_kernel(B: int, L: int):
    n_bblk = B // _BB
    l_per_w = L * n_bblk // _NW
    mesh = plsc.VectorSubcoreMesh(core_axis_name="c", subcore_axis_name="s")

    @functools.partial(
        pl.kernel,
        mesh=mesh,
        out_type=jax.ShapeDtypeStruct((L, OUT_D, B), jnp.float32),
        compiler_params=pltpu.CompilerParams(
            use_tc_tiling_on_sc=True, needs_layout_passes=False),
        scratch_types=[
            pltpu.VMEM((L_PAD, _BB), jnp.int32),
            pltpu.VMEM((L_PAD, _BB), jnp.int32),
            pltpu.VMEM((L_PAD, _BB), jnp.int32),
            pltpu.VMEM((_BB, D_MODEL), jnp.float32),
            pltpu.VMEM((_BB, _BB), jnp.float32),
            pltpu.SemaphoreType.DMA,
        ],
    )
    def k(qk_idx, v_idx, know_idx,
          qk_emb, v_emb, know_emb,
          qk_read, v_read, know_read,
          qk_write, v_write, know_write,
          out,
          qk_iv, v_iv, know_iv,
          buf, slab,
          sem):
        wid = lax.axis_index("s") * _NC + lax.axis_index("c")
        b0 = (wid % n_bblk) * _BB
        l0 = (wid // n_bblk) * l_per_w

        pltpu.sync_copy(qk_idx.at[:, pl.ds(b0, _BB)], qk_iv)
        pltpu.sync_copy(v_idx.at[:, pl.ds(b0, _BB)], v_iv)
        pltpu.sync_copy(know_idx.at[:, pl.ds(b0, _BB)], know_iv)

        iota = lax.iota(jnp.int32, _LANES)

        jobs = [
            (qk_iv, qk_emb, 0, D_B),
            (qk_iv, qk_read, D_B, D_MODEL),
            (qk_iv, qk_write, D_B + D_MODEL, D_MODEL),
            (v_iv, v_emb, POOL_D, D_B),
            (v_iv, v_read, POOL_D + D_B, D_MODEL),
            (v_iv, v_write, POOL_D + D_B + D_MODEL, D_MODEL),
            (know_iv, know_emb, 2 * POOL_D, D_B),
            (know_iv, know_read, 2 * POOL_D + D_B, D_MODEL),
            (know_iv, know_write, 2 * POOL_D + D_B + D_MODEL, D_MODEL),
        ]

        def do_l(i, carry):
            l = l0 + i
            for (iv, tab, d_out, width) in jobs:
                dst = buf if width == D_MODEL else buf.at[:, pl.ds(0, D_PAD)]
                pltpu.async_copy(tab.at[iv.at[l]], dst, sem).wait()

                n_chunk = (width + _BB - 1) // _BB
                chunk_w = min(width, _BB)

                def do_chunk(c, carry2):
                    c0 = c * _BB

                    def do_tb(t_b, carry3):
                        bvec = iota + t_b * _LANES

                        @plsc.parallel_loop(0, _LANES, unroll=8)
                        def do_o(o):
                            rot = (iota + o) & (_LANES - 1)
                            for t_d in range(chunk_w // _LANES):
                                srow = rot + t_d * _LANES
                                dvec = srow + c0
                                v = plsc.load_gather(buf, [bvec, dvec])
                                plsc.store_scatter(slab, [srow, bvec], v)

                        return carry3

                    lax.fori_loop(0, _BB // _LANES, do_tb, 0)

                    pltpu.sync_copy(
                        slab.at[pl.ds(0, chunk_w)],
                        out.at[l, pl.ds(d_out + c0, chunk_w), pl.ds(b0, _BB)])
                    return carry2

                lax.fori_loop(0, n_chunk, do_chunk, 0)
            return carry

        lax.fori_loop(0, l_per_w, do_l, 0)

    return k


def kernel(qk_idx, v_idx, know_idx, qk_emb, v_emb, know_emb,
           qk_read, v_read, know_read, qk_write, v_write, know_write):
    B, L = qk_idx.shape
    ipad = ((0, 0), (0, L_PAD - L))
    epad = ((0, 0), (0, D_PAD - D_B))
    out_p = _make_kernel(B, L)(
        jnp.pad(qk_idx, ipad).T, jnp.pad(v_idx, ipad).T,
        jnp.pad(know_idx, ipad).T,
        jnp.pad(qk_emb, epad), jnp.pad(v_emb, epad), jnp.pad(know_emb, epad),
        qk_read, v_read, know_read,
        qk_write, v_write, know_write)
    return jnp.transpose(out_p, (2, 0, 1))


# speedup vs baseline: 3.0273x; 1.1263x over previous
import functools

import jax
import jax.numpy as jnp
from jax import lax
from jax.experimental import pallas as pl
from jax.experimental.pallas import tpu as pltpu
from jax.experimental.pallas import tpu_sc as plsc

D_MODEL = 768
D_B = 64
D_PAD = 128
L_PAD = 24
POOL_D = D_B + 2 * D_MODEL
OUT_D = 3 * POOL_D

_NC = 2
_NS = 16
_NW = _NC * _NS

_BB = 128
_CW = 64
_LANES = 16


@functools.lru_cache(maxsize=None)
def _make_kernel(B: int, L: int):
    n_bblk = B // _BB
    l_per_w = L * n_bblk // _NW
    mesh = plsc.VectorSubcoreMesh(core_axis_name="c", subcore_axis_name="s")

    @functools.partial(
        pl.kernel,
        mesh=mesh,
        out_type=jax.ShapeDtypeStruct((L, OUT_D, B), jnp.float32),
        compiler_params=pltpu.CompilerParams(
            use_tc_tiling_on_sc=True, needs_layout_passes=False),
        scratch_types=[
            pltpu.VMEM((L_PAD, _BB), jnp.int32),
            pltpu.VMEM((L_PAD, _BB), jnp.int32),
            pltpu.VMEM((L_PAD, _BB), jnp.int32),
            pltpu.VMEM((_BB, D_MODEL), jnp.float32),
            pltpu.VMEM((_CW, _BB), jnp.float32),
            pltpu.VMEM((_CW, _BB), jnp.float32),
            pltpu.SemaphoreType.DMA,
            pltpu.SemaphoreType.DMA,
            pltpu.SemaphoreType.DMA,
        ],
    )
    def k(qk_idx, v_idx, know_idx,
          qk_emb, v_emb, know_emb,
          qk_read, v_read, know_read,
          qk_write, v_write, know_write,
          out,
          qk_iv, v_iv, know_iv,
          buf, slab0, slab1,
          sem, sem0, sem1):
        wid = lax.axis_index("s") * _NC + lax.axis_index("c")
        b0 = (wid % n_bblk) * _BB
        l0 = (wid // n_bblk) * l_per_w

        pltpu.sync_copy(qk_idx.at[:, pl.ds(b0, _BB)], qk_iv)
        pltpu.sync_copy(v_idx.at[:, pl.ds(b0, _BB)], v_iv)
        pltpu.sync_copy(know_idx.at[:, pl.ds(b0, _BB)], know_iv)

        iota = lax.iota(jnp.int32, _LANES)

        jobs = [
            (qk_iv, qk_emb, 0, D_B),
            (qk_iv, qk_read, D_B, D_MODEL),
            (qk_iv, qk_write, D_B + D_MODEL, D_MODEL),
            (v_iv, v_emb, POOL_D, D_B),
            (v_iv, v_read, POOL_D + D_B, D_MODEL),
            (v_iv, v_write, POOL_D + D_B + D_MODEL, D_MODEL),
            (know_iv, know_emb, 2 * POOL_D, D_B),
            (know_iv, know_read, 2 * POOL_D + D_B, D_MODEL),
            (know_iv, know_write, 2 * POOL_D + D_B + D_MODEL, D_MODEL),
        ]

        def do_l(i, carry):
            l = l0 + i
            for (iv, tab, d_out, width) in jobs:
                dst = buf if width == D_MODEL else buf.at[:, pl.ds(0, D_PAD)]
                pltpu.async_copy(tab.at[iv.at[l]], dst, sem).wait()

                def transpose(c0, sl):
                    def do_tb(t_b, carry3):
                        bvec = iota + t_b * _LANES

                        @plsc.parallel_loop(0, _LANES, unroll=8)
                        def do_o(o):
                            rot = (iota + o) & (_LANES - 1)
                            for t_d in range(_CW // _LANES):
                                srow = rot + t_d * _LANES
                                dvec = srow + c0
                                v = plsc.load_gather(buf, [bvec, dvec])
                                plsc.store_scatter(sl, [srow, bvec], v)

                        return carry3

                    lax.fori_loop(0, _BB // _LANES, do_tb, 0)

                if width == D_B:
                    transpose(0, slab0)
                    pltpu.sync_copy(
                        slab0,
                        out.at[l, pl.ds(d_out, _CW), pl.ds(b0, _BB)])
                else:
                    def do_cpair(c2, carry2):
                        for k, (sl, sm) in enumerate(
                                ((slab0, sem0), (slab1, sem1))):
                            c0 = (c2 * 2 + k) * _CW

                            @pl.when(c2 > 0)
                            def _drain():
                                pltpu.make_async_copy(
                                    sl,
                                    out.at[l, pl.ds(d_out + c0 - 2 * _CW, _CW),
                                           pl.ds(b0, _BB)],
                                    sm).wait()

                            transpose(c0, sl)
                            pltpu.async_copy(
                                sl,
                                out.at[l, pl.ds(d_out + c0, _CW),
                                       pl.ds(b0, _BB)],
                                sm)
                        return carry2

                    lax.fori_loop(0, width // (2 * _CW), do_cpair, 0)
                    for sl, sm, off in ((slab0, sem0, width - 2 * _CW),
                                        (slab1, sem1, width - _CW)):
                        pltpu.make_async_copy(
                            sl,
                            out.at[l, pl.ds(d_out + off, _CW), pl.ds(b0, _BB)],
                            sm).wait()
            return carry

        lax.fori_loop(0, l_per_w, do_l, 0)

    return k


def kernel(qk_idx, v_idx, know_idx, qk_emb, v_emb, know_emb,
           qk_read, v_read, know_read, qk_write, v_write, know_write):
    B, L = qk_idx.shape
    ipad = ((0, 0), (0, L_PAD - L))
    epad = ((0, 0), (0, D_PAD - D_B))
    out_p = _make_kernel(B, L)(
        jnp.pad(qk_idx, ipad).T, jnp.pad(v_idx, ipad).T,
        jnp.pad(know_idx, ipad).T,
        jnp.pad(qk_emb, epad), jnp.pad(v_emb, epad), jnp.pad(know_emb, epad),
        qk_read, v_read, know_read,
        qk_write, v_write, know_write)
    return jnp.transpose(out_p, (2, 0, 1))
